# baseline (speedup 1.0000x reference)
import functools

import jax
import jax.numpy as jnp
import numpy as np
from jax import lax
from jax.experimental import pallas as pl
from jax.experimental.pallas import tpu as pltpu
from jax.experimental.pallas import tpu_sc as plsc

_N_ATOMS = 32768
_MN = 8
_N_MOL = _N_ATOMS // _MN
_NN = 64
_F = 4 * _NN

_CBLK = 16384
_NAB = _N_ATOMS // _CBLK

_NC, _NS = 2, 16
_NW = _NC * _NS
_MPW = _N_MOL // _NW
_IPW = _MPW * _MN

_S_NP = np.zeros((_NN, _F), np.float32)
for _n in range(_NN):
    for _c in range(3):
        _S_NP[_n, 4 * _n + _c] = 1.0


def _atom_energy_body(nt_ref, s_ref, sw_ref, out_ref):
    x = nt_ref[...]
    sq = (x * x).reshape(_F, _CBLK)
    r2 = jnp.dot(s_ref[...], sq, preferred_element_type=jnp.float32)
    pred = r2 > 1e-6
    r2s = jnp.where(pred, r2, 1.0)
    r6 = r2s * r2s * r2s
    inv6 = 1.0 / r6
    scale = 2.0 * sw_ref[0, 0]
    pe = jnp.where(pred, scale * (inv6 * inv6 - inv6), 0.0)
    ones = jnp.ones((1, _NN), jnp.float32)
    out_ref[...] = jnp.dot(ones, pe, preferred_element_type=jnp.float32)


def _atom_energies(nt, s_mat, sw):
    return pl.pallas_call(
        _atom_energy_body,
        grid=(_NAB,),
        in_specs=[
            pl.BlockSpec((_NN, 4, _CBLK), lambda i: (0, 0, i)),
            pl.BlockSpec((_NN, _F), lambda i: (0, 0)),
            pl.BlockSpec(memory_space=pltpu.SMEM),
        ],
        out_specs=pl.BlockSpec((1, _CBLK), lambda i: (0, i)),
        out_shape=jax.ShapeDtypeStruct((1, _N_ATOMS), jnp.float32),
    )(nt, s_mat, sw)


def _mol_sum_body(idx_hbm, e_hbm, out_hbm,
                  idx_v, idx2_v, mask_v, rows_v, out_v, sem):
    wid = lax.axis_index("s") * _NC + lax.axis_index("c")
    pltpu.sync_copy(idx_hbm.at[:, pl.ds(wid * _MPW, _MPW)], idx_v)
    descs = []
    for s in range(_MN):
        for k in range(_MPW // 16):
            iv = idx_v[s, pl.ds(k * 16, 16)]
            idx2_v[pl.ds(s * _MPW + k * 16, 16)] = jnp.maximum(iv - 1, 0)
            mask_v[pl.ds(s * _MPW + k * 16, 16)] = jnp.where(iv > 0, 1.0, 0.0)
        descs.append(pltpu.async_copy(
            e_hbm.at[idx2_v.at[pl.ds(s * _MPW, _MPW)]],
            rows_v.at[pl.ds(s * _MPW, _MPW)], sem))
    for d in descs:
        d.wait()
    for c in range(_MPW // 16):
        acc = (rows_v[pl.ds(c * 16, 16)] * mask_v[pl.ds(c * 16, 16)])
        for s in range(1, _MN):
            o = s * _MPW + c * 16
            acc = acc + rows_v[pl.ds(o, 16)] * mask_v[pl.ds(o, 16)]
        out_v[pl.ds(c * 16, 16)] = acc
    pltpu.sync_copy(out_v, out_hbm.at[pl.ds(wid * _MPW, _MPW)])


def _mol_sum(idx_t, e):
    mesh = plsc.VectorSubcoreMesh(core_axis_name="c", subcore_axis_name="s")
    fn = functools.partial(
        pl.kernel,
        out_type=jax.ShapeDtypeStruct((_N_MOL,), jnp.float32),
        mesh=mesh,
        scratch_types=[
            pltpu.VMEM((_MN, _MPW), jnp.int32),
            pltpu.VMEM((_IPW,), jnp.int32),
            pltpu.VMEM((_IPW,), jnp.float32),
            pltpu.VMEM((_IPW,), jnp.float32),
            pltpu.VMEM((_MPW,), jnp.float32),
            pltpu.SemaphoreType.DMA,
        ],
    )(_mol_sum_body)
    return fn(idx_t, e)


def kernel(nlist, positions, box, sample_weight, mol_indices):
    nt = jnp.transpose(nlist, (1, 2, 0))
    idx_t = mol_indices.T
    sw = jnp.reshape(sample_weight, (1, 1)).astype(jnp.float32)
    e = _atom_energies(nt, jnp.asarray(_S_NP), sw)
    return _mol_sum(idx_t, e.reshape(_N_ATOMS))


# speedup vs baseline: 1.0263x; 1.0263x over previous
import functools

import jax
import jax.numpy as jnp
import numpy as np
from jax import lax
from jax.experimental import pallas as pl
from jax.experimental.pallas import tpu as pltpu
from jax.experimental.pallas import tpu_sc as plsc

_N_ATOMS = 32768
_MN = 8
_N_MOL = _N_ATOMS // _MN
_NN = 64
_F = 4 * _NN

_CBLK = 8192
_NAB = _N_ATOMS // _CBLK

_NC, _NS = 2, 16
_NW = _NC * _NS
_MPW = _N_MOL // _NW
_IPW = _MPW * _MN

_S_NP = np.zeros((_NN, _F), np.float32)
for _n in range(_NN):
    for _c in range(3):
        _S_NP[_n, 4 * _n + _c] = 1.0


def _atom_energy_body(nt_ref, s_ref, sw_ref, out_ref):
    x = nt_ref[...]
    sq = (x * x).reshape(_F, _CBLK)
    r2 = jnp.dot(s_ref[...], sq, preferred_element_type=jnp.float32)
    pred = r2 > 1e-6
    r2s = jnp.where(pred, r2, 1.0)
    r6 = r2s * r2s * r2s
    inv6 = 1.0 / r6
    scale = 2.0 * sw_ref[0, 0]
    pe = jnp.where(pred, scale * (inv6 * inv6 - inv6), 0.0)
    ones = jnp.ones((1, _NN), jnp.float32)
    out_ref[...] = jnp.dot(ones, pe, preferred_element_type=jnp.float32)


def _atom_energies(nt, s_mat, sw):
    return pl.pallas_call(
        _atom_energy_body,
        grid=(_NAB,),
        in_specs=[
            pl.BlockSpec((_NN, 4, _CBLK), lambda i: (0, 0, i)),
            pl.BlockSpec((_NN, _F), lambda i: (0, 0)),
            pl.BlockSpec(memory_space=pltpu.SMEM),
        ],
        out_specs=pl.BlockSpec((1, _CBLK), lambda i: (0, i)),
        out_shape=jax.ShapeDtypeStruct((1, _N_ATOMS), jnp.float32),
    )(nt, s_mat, sw)


def _mol_sum_body(idx_hbm, e_hbm, out_hbm,
                  idx_v, idx2_v, mask_v, rows_v, out_v, sem):
    wid = lax.axis_index("s") * _NC + lax.axis_index("c")
    pltpu.sync_copy(idx_hbm.at[:, pl.ds(wid * _MPW, _MPW)], idx_v)
    descs = []
    for s in range(_MN):
        for k in range(_MPW // 16):
            iv = idx_v[s, pl.ds(k * 16, 16)]
            idx2_v[pl.ds(s * _MPW + k * 16, 16)] = jnp.maximum(iv - 1, 0)
            mask_v[pl.ds(s * _MPW + k * 16, 16)] = jnp.where(iv > 0, 1.0, 0.0)
        descs.append(pltpu.async_copy(
            e_hbm.at[idx2_v.at[pl.ds(s * _MPW, _MPW)]],
            rows_v.at[pl.ds(s * _MPW, _MPW)], sem))
    for d in descs:
        d.wait()
    for c in range(_MPW // 16):
        acc = (rows_v[pl.ds(c * 16, 16)] * mask_v[pl.ds(c * 16, 16)])
        for s in range(1, _MN):
            o = s * _MPW + c * 16
            acc = acc + rows_v[pl.ds(o, 16)] * mask_v[pl.ds(o, 16)]
        out_v[pl.ds(c * 16, 16)] = acc
    pltpu.sync_copy(out_v, out_hbm.at[pl.ds(wid * _MPW, _MPW)])


def _mol_sum(idx_t, e):
    mesh = plsc.VectorSubcoreMesh(core_axis_name="c", subcore_axis_name="s")
    fn = functools.partial(
        pl.kernel,
        out_type=jax.ShapeDtypeStruct((_N_MOL,), jnp.float32),
        mesh=mesh,
        scratch_types=[
            pltpu.VMEM((_MN, _MPW), jnp.int32),
            pltpu.VMEM((_IPW,), jnp.int32),
            pltpu.VMEM((_IPW,), jnp.float32),
            pltpu.VMEM((_IPW,), jnp.float32),
            pltpu.VMEM((_MPW,), jnp.float32),
            pltpu.SemaphoreType.DMA,
        ],
    )(_mol_sum_body)
    return fn(idx_t, e)


def kernel(nlist, positions, box, sample_weight, mol_indices):
    nt = jnp.transpose(nlist, (1, 2, 0))
    idx_t = mol_indices.T
    sw = jnp.reshape(sample_weight, (1, 1)).astype(jnp.float32)
    e = _atom_energies(nt, jnp.asarray(_S_NP), sw)
    return _mol_sum(idx_t, e.reshape(_N_ATOMS))


# speedup vs baseline: 1.0384x; 1.0118x over previous
import functools

import jax
import jax.numpy as jnp
import numpy as np
from jax import lax
from jax.experimental import pallas as pl
from jax.experimental.pallas import tpu as pltpu
from jax.experimental.pallas import tpu_sc as plsc

_N_ATOMS = 32768
_MN = 8
_N_MOL = _N_ATOMS // _MN
_NN = 64
_F = 4 * _NN

_NBN = 16
_NNB = _NN // _NBN
_FB = 4 * _NBN

_NC, _NS = 2, 16
_NW = _NC * _NS
_MPW = _N_MOL // _NW
_IPW = _MPW * _MN

_S_NP = np.zeros((_NBN, _FB), np.float32)
for _n in range(_NBN):
    for _c in range(3):
        _S_NP[_n, 4 * _n + _c] = 1.0


def _atom_energy_body(nt_ref, s_ref, sw_ref, out_ref):
    x = nt_ref[...]
    sq = (x * x).reshape(_FB, _N_ATOMS)
    r2 = jnp.dot(s_ref[...], sq, preferred_element_type=jnp.float32)
    pred = r2 > 1e-6
    r2s = jnp.where(pred, r2, 1.0)
    r6 = r2s * r2s * r2s
    inv6 = 1.0 / r6
    scale = 2.0 * sw_ref[0, 0]
    pe = jnp.where(pred, scale * (inv6 * inv6 - inv6), 0.0)
    ones = jnp.ones((1, _NBN), jnp.float32)
    partial = jnp.dot(ones, pe, preferred_element_type=jnp.float32)

    @pl.when(pl.program_id(0) == 0)
    def _():
        out_ref[...] = partial

    @pl.when(pl.program_id(0) != 0)
    def _():
        out_ref[...] = out_ref[...] + partial


def _atom_energies(nt, s_mat, sw):
    return pl.pallas_call(
        _atom_energy_body,
        grid=(_NNB,),
        in_specs=[
            pl.BlockSpec((_NBN, 4, _N_ATOMS), lambda i: (i, 0, 0)),
            pl.BlockSpec((_NBN, _FB), lambda i: (0, 0)),
            pl.BlockSpec(memory_space=pltpu.SMEM),
        ],
        out_specs=pl.BlockSpec((1, _N_ATOMS), lambda i: (0, 0)),
        out_shape=jax.ShapeDtypeStruct((1, _N_ATOMS), jnp.float32),
    )(nt, s_mat, sw)


def _mol_sum_body(idx_hbm, e_hbm, out_hbm,
                  idx_v, idx2_v, mask_v, rows_v, out_v, sem):
    wid = lax.axis_index("s") * _NC + lax.axis_index("c")
    pltpu.sync_copy(idx_hbm.at[:, pl.ds(wid * _MPW, _MPW)], idx_v)
    descs = []
    for s in range(_MN):
        for k in range(_MPW // 16):
            iv = idx_v[s, pl.ds(k * 16, 16)]
            idx2_v[pl.ds(s * _MPW + k * 16, 16)] = jnp.maximum(iv - 1, 0)
            mask_v[pl.ds(s * _MPW + k * 16, 16)] = jnp.where(iv > 0, 1.0, 0.0)
        descs.append(pltpu.async_copy(
            e_hbm.at[idx2_v.at[pl.ds(s * _MPW, _MPW)]],
            rows_v.at[pl.ds(s * _MPW, _MPW)], sem))
    for d in descs:
        d.wait()
    for c in range(_MPW // 16):
        acc = (rows_v[pl.ds(c * 16, 16)] * mask_v[pl.ds(c * 16, 16)])
        for s in range(1, _MN):
            o = s * _MPW + c * 16
            acc = acc + rows_v[pl.ds(o, 16)] * mask_v[pl.ds(o, 16)]
        out_v[pl.ds(c * 16, 16)] = acc
    pltpu.sync_copy(out_v, out_hbm.at[pl.ds(wid * _MPW, _MPW)])


def _mol_sum(idx_t, e):
    mesh = plsc.VectorSubcoreMesh(core_axis_name="c", subcore_axis_name="s")
    fn = functools.partial(
        pl.kernel,
        out_type=jax.ShapeDtypeStruct((_N_MOL,), jnp.float32),
        mesh=mesh,
        scratch_types=[
            pltpu.VMEM((_MN, _MPW), jnp.int32),
            pltpu.VMEM((_IPW,), jnp.int32),
            pltpu.VMEM((_IPW,), jnp.float32),
            pltpu.VMEM((_IPW,), jnp.float32),
            pltpu.VMEM((_MPW,), jnp.float32),
            pltpu.SemaphoreType.DMA,
        ],
    )(_mol_sum_body)
    return fn(idx_t, e)


def kernel(nlist, positions, box, sample_weight, mol_indices):
    nt = jnp.transpose(nlist, (1, 2, 0))
    idx_t = mol_indices.T
    sw = jnp.reshape(sample_weight, (1, 1)).astype(jnp.float32)
    e = _atom_energies(nt, jnp.asarray(_S_NP), sw)
    return _mol_sum(idx_t, e.reshape(_N_ATOMS))
